# baseline (speedup 1.0000x reference)
import jax
import jax.numpy as jnp
from jax import lax
from jax.experimental import pallas as pl
from jax.experimental.pallas import tpu as pltpu

_IN_F = 4096
_OUT_F = 4096
_SPLIT = [64, 96]
_TWO_POW_112 = 5.192296858534828e33


def _f16_bits_to_f32(h16):
    h = h16.astype(jnp.int32)
    sign = jnp.bitwise_and(h, jnp.int32(-2147483648))
    expmant = jnp.left_shift(jnp.bitwise_and(h, jnp.int32(0x7FFF)), 13)
    f = lax.bitcast_convert_type(jnp.bitwise_or(sign, expmant), jnp.float32)
    return f * jnp.float32(_TWO_POW_112)


def _f32_to_f16_bits(f):
    small = f * jnp.float32(1.0 / _TWO_POW_112)
    b = lax.bitcast_convert_type(small, jnp.int32)
    sign16 = jnp.bitwise_and(jnp.right_shift(b, 16), jnp.int32(0x8000))
    mag = jnp.bitwise_and(b, jnp.int32(0x7FFFFFFF))
    round_bias = jnp.int32(0xFFF) + jnp.bitwise_and(jnp.right_shift(mag, 13), 1)
    mag = mag + round_bias
    h = jnp.bitwise_or(sign16, jnp.bitwise_and(jnp.right_shift(mag, 13), jnp.int32(0x7FFF)))
    return h.astype(jnp.int16)


def _body(wids_ref, x_ref, at_ref, b_ref, o_ref):
    at = _f16_bits_to_f32(at_ref[...])
    xv = _f16_bits_to_f32(x_ref[...])
    u = lax.dot_general(
        xv, at, (((1,), (1,)), ((), ())), preferred_element_type=jnp.float32
    )
    u = _f16_bits_to_f32(_f32_to_f16_bits(u))
    bm = _f16_bits_to_f32(b_ref[...])
    y = lax.dot_general(
        u, bm, (((1,), (0,)), ((), ())), preferred_element_type=jnp.float32
    )
    o_ref[...] = _f32_to_f16_bits(2.0 * y)


def _lora_group(x3d_i16, wids, at_tab_i16, b_tab_i16):
    b = x3d_i16.shape[0]
    r = at_tab_i16.shape[1]
    grid_spec = pltpu.PrefetchScalarGridSpec(
        num_scalar_prefetch=1,
        grid=(b,),
        in_specs=[
            pl.BlockSpec((None, 1, _IN_F), lambda i, w: (i, 0, 0)),
            pl.BlockSpec((None, r, _IN_F), lambda i, w: (w[i], 0, 0)),
            pl.BlockSpec((None, r, _OUT_F), lambda i, w: (w[i], 0, 0)),
        ],
        out_specs=pl.BlockSpec((None, 1, _OUT_F), lambda i, w: (i, 0, 0)),
    )
    return pl.pallas_call(
        _body,
        grid_spec=grid_spec,
        out_shape=jax.ShapeDtypeStruct((b, 1, _OUT_F), jnp.int16),
        compiler_params=pltpu.CompilerParams(
            dimension_semantics=("arbitrary",),
        ),
    )(wids, x3d_i16, at_tab_i16, b_tab_i16)


def _i16(v):
    return lax.bitcast_convert_type(v, jnp.int16)


def kernel(x, wids_0, wids_1, wids_2, lora_A_0, lora_B_0, lora_A_1, lora_B_1, lora_A_2, lora_B_2):
    xs = jnp.split(_i16(x), _SPLIT, axis=0)
    ys = [
        _lora_group(xs[0], wids_0, _i16(jnp.swapaxes(lora_A_0, 1, 2)), _i16(lora_B_0)),
        _lora_group(xs[1], wids_1, _i16(jnp.swapaxes(lora_A_1, 1, 2)), _i16(lora_B_1)),
        _lora_group(xs[2], wids_2, _i16(jnp.swapaxes(lora_A_2, 1, 2)), _i16(lora_B_2)),
    ]
    out = jnp.concatenate(ys, axis=0)
    return lax.bitcast_convert_type(out, jnp.float16)
